# baseline (speedup 1.0000x reference)
import functools

import jax
import jax.numpy as jnp
from jax import lax
from jax.experimental import pallas as pl
from jax.experimental.pallas import tpu as pltpu
from jax.experimental.pallas import tpu_sc as plsc

_D = 1024
_NC = 2
_NS = 16
_NW = _NC * _NS
_CHUNK = 56


@functools.cache
def _build(batch, seq):
    n_total = batch * seq
    b_per_w = n_total // _NW
    w_per_row = seq // b_per_w
    lens = [_CHUNK] * (b_per_w // _CHUNK)
    if b_per_w % _CHUNK:
        lens.append(b_per_w % _CHUNK)
    offs = [sum(lens[:i]) for i in range(len(lens))]
    n_chunks = len(lens)
    mesh = plsc.VectorSubcoreMesh(
        core_axis_name="c", subcore_axis_name="s",
        num_cores=_NC, num_subcores=_NS)

    @functools.partial(
        pl.kernel,
        out_type=jax.ShapeDtypeStruct((batch, seq, _D), jnp.float32),
        mesh=mesh,
        scratch_types=[
            pltpu.VMEM((b_per_w,), jnp.int32),
            [pltpu.VMEM((_CHUNK, _D), jnp.float32) for _ in range(2)],
            [pltpu.SemaphoreType.DMA for _ in range(2)],
            [pltpu.SemaphoreType.DMA for _ in range(2)],
        ],
    )
    def gather_kernel(idx_hbm, table_hbm, out_hbm, idx_v, bufs, sgs, sos):
        wid = lax.axis_index("s") * _NC + lax.axis_index("c")
        row = wid // w_per_row
        col = (wid % w_per_row) * b_per_w
        pltpu.sync_copy(idx_hbm.at[row, pl.ds(col, b_per_w)], idx_v)

        def gather(j):
            return pltpu.async_copy(
                table_hbm.at[idx_v.at[pl.ds(offs[j], lens[j])]],
                bufs[j % 2].at[pl.ds(0, lens[j])], sgs[j % 2])

        def put(j):
            return pltpu.async_copy(
                bufs[j % 2].at[pl.ds(0, lens[j])],
                out_hbm.at[row, pl.ds(col + offs[j], lens[j])],
                sos[j % 2])

        gathers = [None] * n_chunks
        outs = [None] * n_chunks
        gathers[0] = gather(0)
        for j in range(n_chunks):
            if j + 1 < n_chunks:
                if j >= 1:
                    outs[j - 1].wait()
                gathers[j + 1] = gather(j + 1)
            gathers[j].wait()
            outs[j] = put(j)
        outs[n_chunks - 2].wait()
        outs[n_chunks - 1].wait()

    return gather_kernel


def kernel(positions, positional_encoding_matrix):
    b, s = positions.shape
    return _build(b, s)(positions, positional_encoding_matrix)
